# Initial kernel scaffold; baseline (speedup 1.0000x reference)
import jax
import jax.numpy as jnp
from jax.experimental import pallas as pl


def kernel(input, table):
    raise NotImplementedError("write your pallas kernel here")


# speedup vs baseline: 1.2704x; 1.2704x over previous
import functools

import jax
import jax.numpy as jnp
from jax import lax
from jax.experimental import pallas as pl
from jax.experimental.pallas import tpu as pltpu
from jax.experimental.pallas import tpu_sc as plsc

_NC = 2
_NS = 16
_NW = _NC * _NS
_CHUNK = 128


@functools.cache
def _build(n, dp):
    n_per_w = n // _NW
    n_chunks = n_per_w // _CHUNK
    assert n_per_w * _NW == n and n_chunks * _CHUNK == n_per_w

    mesh = plsc.VectorSubcoreMesh(core_axis_name="c", subcore_axis_name="s")

    @functools.partial(
        pl.kernel,
        mesh=mesh,
        out_type=jax.ShapeDtypeStruct((n, dp), jnp.float32),
        scratch_types=[
            pltpu.VMEM((n_chunks, _CHUNK), jnp.int32),
            pltpu.VMEM((_CHUNK, dp), jnp.float32),
            pltpu.SemaphoreType.DMA,
        ],
    )
    def gather_kernel(idx_hbm, table_hbm, out_hbm, idx_v, buf0, gsem):
        wid = lax.axis_index("s") * _NC + lax.axis_index("c")
        base = wid * n_per_w
        pltpu.sync_copy(idx_hbm.at[wid], idx_v)

        def body(c, carry):
            pltpu.async_copy(table_hbm.at[idx_v.at[c]], buf0, gsem).wait()
            pltpu.sync_copy(buf0, out_hbm.at[pl.ds(base + c * _CHUNK, _CHUNK)])
            return carry

        lax.fori_loop(0, n_chunks, body, 0)

    return gather_kernel


def kernel(input, table):
    b, s = input.shape
    _, d = table.shape
    n = b * s
    dp = (d + 127) // 128 * 128
    idx = input.reshape(_NW, (n // _NW) // _CHUNK, _CHUNK).astype(jnp.int32)
    table_p = jnp.pad(table, ((0, 0), (0, dp - d)))
    out = _build(n, dp)(idx, table_p)
    return out[:, :d].reshape(b, s, d)


# speedup vs baseline: 1.3093x; 1.0307x over previous
import functools

import jax
import jax.numpy as jnp
from jax import lax
from jax.experimental import pallas as pl
from jax.experimental.pallas import tpu as pltpu
from jax.experimental.pallas import tpu_sc as plsc

_NC = 2
_NS = 16
_NW = _NC * _NS
_CHUNK = 128


@functools.cache
def _build(n, dp):
    n_per_w = n // _NW
    n_chunks = n_per_w // _CHUNK
    assert n_per_w * _NW == n and n_chunks * _CHUNK == n_per_w

    mesh = plsc.VectorSubcoreMesh(core_axis_name="c", subcore_axis_name="s")

    @functools.partial(
        pl.kernel,
        mesh=mesh,
        out_type=jax.ShapeDtypeStruct((n, dp), jnp.float32),
        scratch_types=[
            pltpu.VMEM((n_chunks, _CHUNK), jnp.int32),
            pltpu.VMEM((_CHUNK, dp), jnp.float32),
            pltpu.VMEM((_CHUNK, dp), jnp.float32),
            pltpu.SemaphoreType.DMA,
        ],
    )
    def gather_kernel(idx_hbm, table_hbm, out_hbm, idx_v, buf0, buf1, gsem):
        wid = lax.axis_index("s") * _NC + lax.axis_index("c")
        base = wid * n_per_w
        pltpu.sync_copy(idx_hbm.at[wid], idx_v)

        def start(c, buf):
            pltpu.async_copy(table_hbm.at[idx_v.at[c]], buf, gsem)

        def wait(buf):
            pltpu.make_async_copy(table_hbm.at[idx_v.at[0]], buf, gsem).wait()

        def store(c, buf):
            pltpu.sync_copy(buf, out_hbm.at[pl.ds(base + c * _CHUNK, _CHUNK)])

        start(0, buf0)

        def body(c2, carry):
            c0 = 2 * c2
            wait(buf0)
            start(c0 + 1, buf1)
            store(c0, buf0)
            wait(buf1)
            start(jnp.minimum(c0 + 2, n_chunks - 1), buf0)
            store(c0 + 1, buf1)
            return carry

        lax.fori_loop(0, n_chunks // 2, body, 0)
        wait(buf0)

    return gather_kernel


def kernel(input, table):
    b, s = input.shape
    _, d = table.shape
    n = b * s
    dp = (d + 127) // 128 * 128
    idx = input.reshape(_NW, (n // _NW) // _CHUNK, _CHUNK).astype(jnp.int32)
    table_p = jnp.pad(table, ((0, 0), (0, dp - d)))
    out = _build(n, dp)(idx, table_p)
    return out[:, :d].reshape(b, s, d)


# speedup vs baseline: 1.4770x; 1.1280x over previous
import functools

import jax
import jax.numpy as jnp
from jax import lax
from jax.experimental import pallas as pl
from jax.experimental.pallas import tpu as pltpu
from jax.experimental.pallas import tpu_sc as plsc

_NC = 2
_NS = 16
_NW = _NC * _NS
_CHUNK = 128
_DMAIN = 256


@functools.cache
def _build(n, d, dp):
    n_per_w = n // _NW
    n_chunks = n_per_w // _CHUNK
    assert n_per_w * _NW == n and n_chunks * _CHUNK == n_per_w
    assert n_chunks % 2 == 0
    dt = dp - _DMAIN

    mesh = plsc.VectorSubcoreMesh(core_axis_name="c", subcore_axis_name="s")

    @functools.partial(
        pl.kernel,
        mesh=mesh,
        out_type=jax.ShapeDtypeStruct((n, dp), jnp.float32),
        scratch_types=[
            pltpu.VMEM((n_chunks, _CHUNK), jnp.int32),
            pltpu.VMEM((_CHUNK, dp), jnp.float32),
            pltpu.VMEM((_CHUNK, dp), jnp.float32),
            pltpu.SemaphoreType.DMA,
        ],
    )
    def gather_kernel(idx_hbm, table_hbm, tail_hbm, out_hbm, idx_v, buf0, buf1, gsem):
        wid = lax.axis_index("s") * _NC + lax.axis_index("c")
        base = wid * n_per_w
        pltpu.sync_copy(idx_hbm.at[wid], idx_v)

        def start(c, buf):
            pltpu.async_copy(
                table_hbm.at[idx_v.at[c], pl.ds(0, _DMAIN)],
                buf.at[:, pl.ds(0, _DMAIN)],
                gsem,
            )
            pltpu.async_copy(
                tail_hbm.at[idx_v.at[c]],
                buf.at[:, pl.ds(_DMAIN, dt)],
                gsem,
            )

        def wait(buf):
            pltpu.make_async_copy(
                table_hbm.at[idx_v.at[0], pl.ds(0, _DMAIN)],
                buf.at[:, pl.ds(0, _DMAIN)],
                gsem,
            ).wait()
            pltpu.make_async_copy(
                tail_hbm.at[idx_v.at[0]],
                buf.at[:, pl.ds(_DMAIN, dt)],
                gsem,
            ).wait()

        def store(c, buf):
            pltpu.sync_copy(buf, out_hbm.at[pl.ds(base + c * _CHUNK, _CHUNK)])

        start(0, buf0)

        def body(c2, carry):
            c0 = 2 * c2
            wait(buf0)
            start(c0 + 1, buf1)
            store(c0, buf0)
            wait(buf1)
            start(jnp.minimum(c0 + 2, n_chunks - 1), buf0)
            store(c0 + 1, buf1)
            return carry

        lax.fori_loop(0, n_chunks // 2, body, 0)
        wait(buf0)

    return gather_kernel


def kernel(input, table):
    b, s = input.shape
    v, d = table.shape
    n = b * s
    dp = (d + 127) // 128 * 128
    idx = input.reshape(_NW, (n // _NW) // _CHUNK, _CHUNK).astype(jnp.int32)
    tail = jnp.pad(table[:, _DMAIN:], ((0, 0), (0, dp - d)))
    out = _build(n, d, dp)(idx, table, tail)
    return out[:, :d].reshape(b, s, d)


# speedup vs baseline: 1.4779x; 1.0006x over previous
import functools

import jax
import jax.numpy as jnp
from jax import lax
from jax.experimental import pallas as pl
from jax.experimental.pallas import tpu as pltpu
from jax.experimental.pallas import tpu_sc as plsc

_NC = 2
_NS = 16
_NW = _NC * _NS
_CHUNK = 128
_DMAIN = 256
_DTAIL = 128
_LANES = 16


@functools.cache
def _build(n, d):
    n_per_w = n // _NW
    n_chunks = n_per_w // _CHUNK
    assert n_per_w * _NW == n and n_chunks * _CHUNK == n_per_w
    assert n_chunks % 2 == 0
    dt = d - _DMAIN

    mesh = plsc.VectorSubcoreMesh(core_axis_name="c", subcore_axis_name="s")

    @functools.partial(
        pl.kernel,
        mesh=mesh,
        compiler_params=pltpu.CompilerParams(needs_layout_passes=False),
        out_type=jax.ShapeDtypeStruct((n, d), jnp.float32),
        scratch_types=[
            pltpu.VMEM((_CHUNK,), jnp.int32),
            pltpu.VMEM((_CHUNK,), jnp.int32),
            pltpu.VMEM((_CHUNK, d), jnp.float32),
            pltpu.VMEM((_CHUNK, d), jnp.float32),
            pltpu.VMEM((_CHUNK, _DTAIL), jnp.float32),
            pltpu.SemaphoreType.DMA,
            pltpu.SemaphoreType.DMA,
        ],
    )
    def gather_kernel(idx_hbm, table_hbm, tail_hbm, out_hbm,
                      idxb0, idxb1, bufa0, bufa1, bufb, isem, gsem):
        wid = lax.axis_index("s") * _NC + lax.axis_index("c")
        base = wid * n_per_w

        def load_idx(c, idxb):
            pltpu.async_copy(idx_hbm.at[wid, c], idxb, isem)

        def wait_idx(idxb):
            pltpu.make_async_copy(idx_hbm.at[0, 0], idxb, isem).wait()

        def start_main(idxb, bufa):
            pltpu.async_copy(
                table_hbm.at[idxb, pl.ds(0, _DMAIN)],
                bufa.at[:, pl.ds(0, _DMAIN)],
                gsem,
            )

        def start_tail(idxb):
            pltpu.async_copy(tail_hbm.at[idxb], bufb, gsem)

        def wait_main(bufa):
            pltpu.make_async_copy(
                table_hbm.at[idxb0, pl.ds(0, _DMAIN)],
                bufa.at[:, pl.ds(0, _DMAIN)],
                gsem,
            ).wait()

        def wait_tail():
            pltpu.make_async_copy(tail_hbm.at[idxb0], bufb, gsem).wait()

        lanes = lax.iota(jnp.int32, _LANES)
        tail_cols = _DMAIN + 2 * _LANES + lanes
        tail_mask = lanes < dt - 2 * _LANES

        def repack(bufa):
            def row(r, carry):
                for k in (0, _LANES):
                    bufa[r, pl.ds(_DMAIN + k, _LANES)] = bufb[r, pl.ds(k, _LANES)]
                x = bufb[r, pl.ds(2 * _LANES, _LANES)]
                plsc.store_scatter(
                    bufa,
                    [jnp.full((_LANES,), r, jnp.int32), tail_cols],
                    x,
                    mask=tail_mask,
                )
                return carry

            lax.fori_loop(0, _CHUNK, row, 0, unroll=4)

        def store(c, bufa):
            pltpu.sync_copy(bufa, out_hbm.at[pl.ds(base + c * _CHUNK, _CHUNK)])

        pltpu.sync_copy(idx_hbm.at[wid, 0], idxb0)
        start_main(idxb0, bufa0)
        start_tail(idxb0)
        load_idx(1, idxb1)

        def body(c2, carry):
            c0 = 2 * c2
            wait_main(bufa0)
            wait_tail()
            wait_idx(idxb1)
            start_main(idxb1, bufa1)
            load_idx(jnp.minimum(c0 + 2, n_chunks - 1), idxb0)
            repack(bufa0)
            start_tail(idxb1)
            store(c0, bufa0)
            wait_main(bufa1)
            wait_tail()
            wait_idx(idxb0)
            start_main(idxb0, bufa0)
            load_idx(jnp.minimum(c0 + 3, n_chunks - 1), idxb1)
            repack(bufa1)
            start_tail(idxb0)
            store(c0 + 1, bufa1)
            return carry

        lax.fori_loop(0, n_chunks // 2, body, 0)
        wait_main(bufa0)
        wait_tail()
        wait_idx(idxb1)

    return gather_kernel


def kernel(input, table):
    b, s = input.shape
    v, d = table.shape
    n = b * s
    idx = input.reshape(_NW, (n // _NW) // _CHUNK, _CHUNK).astype(jnp.int32)
    tail = jnp.pad(table[:, _DMAIN:], ((0, 0), (0, _DTAIL - (d - _DMAIN))))
    out = _build(n, d)(idx, table, tail)
    return out.reshape(b, s, d)
